# Initial kernel scaffold; baseline (speedup 1.0000x reference)
import jax
import jax.numpy as jnp
from jax.experimental import pallas as pl


def kernel(x_cpd, x_ko, edge_index_cpd_to_ko, edge_index_ko_to_cpd, lin0_W_cpd, lin0_b_cpd, lin0_W_ko, lin0_b_ko, L0_k_W_cpd, L0_k_b_cpd, L0_q_W_cpd, L0_q_b_cpd, L0_v_W_cpd, L0_v_b_cpd, L0_a_W_cpd, L0_a_b_cpd, L0_skip_cpd, L0_k_W_ko, L0_k_b_ko, L0_q_W_ko, L0_q_b_ko, L0_v_W_ko, L0_v_b_ko, L0_a_W_ko, L0_a_b_ko, L0_skip_ko, L0_arel_c2k, L0_mrel_c2k, L0_prel_c2k, L0_arel_k2c, L0_mrel_k2c, L0_prel_k2c, L1_k_W_cpd, L1_k_b_cpd, L1_q_W_cpd, L1_q_b_cpd, L1_v_W_cpd, L1_v_b_cpd, L1_a_W_cpd, L1_a_b_cpd, L1_skip_cpd, L1_k_W_ko, L1_k_b_ko, L1_q_W_ko, L1_q_b_ko, L1_v_W_ko, L1_v_b_ko, L1_a_W_ko, L1_a_b_ko, L1_skip_ko, L1_arel_c2k, L1_mrel_c2k, L1_prel_c2k, L1_arel_k2c, L1_mrel_k2c, L1_prel_k2c, lin_out_W, lin_out_b):
    raise NotImplementedError("write your pallas kernel here")


# speedup vs baseline: 33.5579x; 33.5579x over previous
import functools
import math

import jax
import jax.numpy as jnp
from jax import lax
from jax.experimental import pallas as pl
from jax.experimental.pallas import tpu as pltpu, tpu_sc as plsc
from jax.scipy.linalg import block_diag

N_REAL = 25000
D_IN = 128
HID = 128
HEADS = 4
DH = HID // HEADS
OUT = 64
E_REAL = 300000

NP = 25088
HALF = 12544
ACC_ROWS = 12672
EP = 303104
EW = 16

_sc_mesh = plsc.VectorSubcoreMesh(core_axis_name="c", subcore_axis_name="s")
_sc_params = pltpu.CompilerParams(use_tc_tiling_on_sc=False,
                                  needs_layout_passes=False)


def _linear_body(act, x_ref, w_ref, b_ref, o_ref):
    y = jnp.dot(x_ref[...], w_ref[...], preferred_element_type=jnp.float32)
    y = y + b_ref[...]
    if act == "relu":
        y = jnp.maximum(y, 0.0)
    o_ref[...] = y


def _linear(x, w, b, act, block, m_out=None):
    m = m_out if m_out is not None else x.shape[0]
    k = x.shape[1]
    n = w.shape[1]
    return pl.pallas_call(
        functools.partial(_linear_body, act),
        grid=(m // block,),
        in_specs=[
            pl.BlockSpec((block, k), lambda i: (i, 0)),
            pl.BlockSpec((k, n), lambda i: (0, 0)),
            pl.BlockSpec((1, n), lambda i: (0, 0)),
        ],
        out_specs=pl.BlockSpec((block, n), lambda i: (i, 0)),
        out_shape=jax.ShapeDtypeStruct((m, n), jnp.float32),
    )(x, w, b.reshape(1, n))


def _proj_body(x_ref, w_ref, b_ref, q_ref, kv_ref):
    y = jnp.dot(x_ref[...], w_ref[...], preferred_element_type=jnp.float32)
    y = y + b_ref[...]
    q_ref[...] = y[:, :HID]
    kv_ref[...] = y[:, HID:]


def _proj(h, wcat, bcat, block=1568):
    return pl.pallas_call(
        _proj_body,
        grid=(NP // block,),
        in_specs=[
            pl.BlockSpec((block, HID), lambda i: (i, 0)),
            pl.BlockSpec((HID, 3 * HID), lambda i: (0, 0)),
            pl.BlockSpec((1, 3 * HID), lambda i: (0, 0)),
        ],
        out_specs=[
            pl.BlockSpec((block, HID), lambda i: (i, 0)),
            pl.BlockSpec((block, 2 * HID), lambda i: (i, 0)),
        ],
        out_shape=[
            jax.ShapeDtypeStruct((NP, HID), jnp.float32),
            jax.ShapeDtypeStruct((NP, 2 * HID), jnp.float32),
        ],
    )(h, wcat, bcat.reshape(1, 3 * HID))


def _msg_body(q_ref, kv_ref, hsum_ref, sel_ref, v_ref, e_ref):
    q = q_ref[...]
    kv = kv_ref[...]
    k = kv[:, :HID]
    v = kv[:, HID:]
    p = q * k
    s128 = jnp.dot(p, hsum_ref[...], preferred_element_type=jnp.float32)
    e128 = jnp.exp(s128)
    e4 = jnp.dot(e128, sel_ref[...], preferred_element_type=jnp.float32)
    z = jnp.zeros((q.shape[0], EW - HEADS), jnp.float32)
    v_ref[...] = v * e128
    e_ref[...] = jnp.concatenate([e4, z], axis=1)


def _msg(qg, kvg, hsum, sel, block=2048):
    return pl.pallas_call(
        _msg_body,
        grid=(EP // block,),
        in_specs=[
            pl.BlockSpec((block, HID), lambda i: (i, 0)),
            pl.BlockSpec((block, 2 * HID), lambda i: (i, 0)),
            pl.BlockSpec((HID, HID), lambda i: (0, 0)),
            pl.BlockSpec((HID, HEADS), lambda i: (0, 0)),
        ],
        out_specs=[
            pl.BlockSpec((block, HID), lambda i: (i, 0)),
            pl.BlockSpec((block, EW), lambda i: (i, 0)),
        ],
        out_shape=[
            jax.ShapeDtypeStruct((EP, HID), jnp.float32),
            jax.ShapeDtypeStruct((EP, EW), jnp.float32),
        ],
    )(qg, kvg, hsum, sel)


def _update_body(num_ref, den_ref, bc_ref, h_ref, w_ref, b_ref, s_ref, o_ref):
    den = jnp.dot(den_ref[...], bc_ref[...],
                  preferred_element_type=jnp.float32)
    agg = num_ref[...] / (den + 1e-16)
    g = jax.nn.gelu(agg)
    o = jnp.dot(g, w_ref[...], preferred_element_type=jnp.float32) + b_ref[...]
    a = jax.nn.sigmoid(s_ref[0, 0])
    o_ref[...] = a * o + (1.0 - a) * h_ref[...]


def _update(num, den, bc, h, w, b, skip, block=1568):
    return pl.pallas_call(
        _update_body,
        grid=(NP // block,),
        in_specs=[
            pl.BlockSpec((block, HID), lambda i: (i, 0)),
            pl.BlockSpec((block, EW), lambda i: (i, 0)),
            pl.BlockSpec((EW, HID), lambda i: (0, 0)),
            pl.BlockSpec((block, HID), lambda i: (i, 0)),
            pl.BlockSpec((HID, HID), lambda i: (0, 0)),
            pl.BlockSpec((1, HID), lambda i: (0, 0)),
            pl.BlockSpec((1, 1), lambda i: (0, 0)),
        ],
        out_specs=pl.BlockSpec((block, HID), lambda i: (i, 0)),
        out_shape=jax.ShapeDtypeStruct((NP, HID), jnp.float32),
    )(num, den, bc, h, w, b.reshape(1, HID), skip.reshape(1, 1))


def _gather(q_tab, kv_tab, dst_g, src_g):
    epw = EP // 32
    ch = 128

    @functools.partial(
        pl.kernel,
        mesh=_sc_mesh,
        compiler_params=_sc_params,
        out_type=[
            jax.ShapeDtypeStruct((EP, HID), jnp.float32),
            jax.ShapeDtypeStruct((EP, 2 * HID), jnp.float32),
        ],
        scratch_types=[
            pltpu.VMEM((ch,), jnp.int32),
            pltpu.VMEM((ch,), jnp.int32),
            pltpu.VMEM((ch, HID), jnp.float32),
            pltpu.VMEM((ch, 2 * HID), jnp.float32),
            pltpu.SemaphoreType.DMA,
            pltpu.SemaphoreType.DMA,
        ],
    )
    def k(qt, kvt, dg, sg, qg, kvg, di, si, qb, kvb, s1, s2):
        wid = lax.axis_index("s") * 2 + lax.axis_index("c")
        base = wid * epw

        def body(ci, carry):
            off = base + ci * ch
            pltpu.sync_copy(dg.at[pl.ds(off, ch)], di)
            pltpu.sync_copy(sg.at[pl.ds(off, ch)], si)
            cq = pltpu.async_copy(qt.at[di], qb, s1)
            ck = pltpu.async_copy(kvt.at[si], kvb, s2)
            cq.wait()
            ck.wait()
            pltpu.sync_copy(qb, qg.at[pl.ds(off, ch)])
            pltpu.sync_copy(kvb, kvg.at[pl.ds(off, ch)])
            return carry

        lax.fori_loop(0, epw // ch, body, 0)

    return k(q_tab, kv_tab, dst_g, src_g)


def _scatter(msg, dst_s, zeros_acc, width):
    ept = EP // 16
    ch = 128
    nch = ept // ch
    zrows = ACC_ROWS // 16
    rpt = HALF // 16

    @functools.partial(
        pl.kernel,
        mesh=_sc_mesh,
        compiler_params=_sc_params,
        out_type=jax.ShapeDtypeStruct((NP, width), jnp.float32),
        scratch_types=[
            pltpu.VMEM((ch,), jnp.int32),
            pltpu.VMEM((ch,), jnp.int32),
            pltpu.VMEM((ch, width), jnp.float32),
            pltpu.VMEM_SHARED((ACC_ROWS, width), jnp.float32),
            pltpu.SemaphoreType.DMA,
        ],
    )
    def k(mh, dh, zh, agg, di, li, mb, acc, sem):
        cid = lax.axis_index("c")
        sid = lax.axis_index("s")
        pltpu.sync_copy(zh.at[pl.ds(sid * zrows, zrows)],
                        acc.at[pl.ds(sid * zrows, zrows)])
        plsc.subcore_barrier()

        nbase = cid * HALF
        ebase = sid * ept

        def body(ci, carry):
            off = ebase + ci * ch
            pltpu.sync_copy(dh.at[pl.ds(off, ch)], di)
            pltpu.sync_copy(mh.at[pl.ds(off, ch)], mb)
            for g in range(ch // 16):
                dv = di[pl.ds(g * 16, 16)]
                l = dv - nbase
                ok = (l >= 0) & (l < HALF)
                li[pl.ds(g * 16, 16)] = jnp.where(ok, l, HALF)
            pltpu.sync_copy(mb, acc.at[li], add=True)
            return carry

        lax.fori_loop(0, nch, body, 0)
        plsc.subcore_barrier()

        pltpu.sync_copy(acc.at[pl.ds(sid * rpt, rpt)],
                        agg.at[pl.ds(nbase + sid * rpt, rpt)])

    return k(msg, dst_s, zeros_acc)


def _pad_nodes(x):
    return jnp.concatenate(
        [x, jnp.zeros((NP - N_REAL, x.shape[1]), x.dtype)], axis=0)


def _pad_edges(ei):
    src, dst = ei[0], ei[1]
    pad = EP - E_REAL
    src_g = jnp.concatenate([src, jnp.zeros((pad,), jnp.int32)])
    dst_g = jnp.concatenate([dst, jnp.zeros((pad,), jnp.int32)])
    dst_s = jnp.concatenate([dst, jnp.full((pad,), NP, jnp.int32)])
    return src_g, dst_g, dst_s


def _combined_weights(wq, bq, wk, bk, wv, bv, arel, mrel, prel):
    scale = prel / math.sqrt(DH)
    arel_s = arel * scale[:, None, None]
    bd_a = block_diag(*[arel_s[h] for h in range(HEADS)])
    bd_m = block_diag(*[mrel[h] for h in range(HEADS)])
    wcat = jnp.concatenate([wq, wk @ bd_a, wv @ bd_m], axis=1)
    bcat = jnp.concatenate([bq, bk @ bd_a, bv @ bd_m], axis=0)
    return wcat, bcat


def kernel(x_cpd, x_ko, edge_index_cpd_to_ko, edge_index_ko_to_cpd,
           lin0_W_cpd, lin0_b_cpd, lin0_W_ko, lin0_b_ko,
           L0_k_W_cpd, L0_k_b_cpd, L0_q_W_cpd, L0_q_b_cpd, L0_v_W_cpd,
           L0_v_b_cpd, L0_a_W_cpd, L0_a_b_cpd, L0_skip_cpd,
           L0_k_W_ko, L0_k_b_ko, L0_q_W_ko, L0_q_b_ko, L0_v_W_ko, L0_v_b_ko,
           L0_a_W_ko, L0_a_b_ko, L0_skip_ko,
           L0_arel_c2k, L0_mrel_c2k, L0_prel_c2k,
           L0_arel_k2c, L0_mrel_k2c, L0_prel_k2c,
           L1_k_W_cpd, L1_k_b_cpd, L1_q_W_cpd, L1_q_b_cpd, L1_v_W_cpd,
           L1_v_b_cpd, L1_a_W_cpd, L1_a_b_cpd, L1_skip_cpd,
           L1_k_W_ko, L1_k_b_ko, L1_q_W_ko, L1_q_b_ko, L1_v_W_ko, L1_v_b_ko,
           L1_a_W_ko, L1_a_b_ko, L1_skip_ko,
           L1_arel_c2k, L1_mrel_c2k, L1_prel_c2k,
           L1_arel_k2c, L1_mrel_k2c, L1_prel_k2c,
           lin_out_W, lin_out_b):
    L = {0: {}, 1: {}}
    L[0].update(k_W_cpd=L0_k_W_cpd, k_b_cpd=L0_k_b_cpd, q_W_cpd=L0_q_W_cpd,
                q_b_cpd=L0_q_b_cpd, v_W_cpd=L0_v_W_cpd, v_b_cpd=L0_v_b_cpd,
                a_W_cpd=L0_a_W_cpd, a_b_cpd=L0_a_b_cpd, skip_cpd=L0_skip_cpd,
                k_W_ko=L0_k_W_ko, k_b_ko=L0_k_b_ko, q_W_ko=L0_q_W_ko,
                q_b_ko=L0_q_b_ko, v_W_ko=L0_v_W_ko, v_b_ko=L0_v_b_ko,
                a_W_ko=L0_a_W_ko, a_b_ko=L0_a_b_ko, skip_ko=L0_skip_ko,
                arel_c2k=L0_arel_c2k, mrel_c2k=L0_mrel_c2k,
                prel_c2k=L0_prel_c2k, arel_k2c=L0_arel_k2c,
                mrel_k2c=L0_mrel_k2c, prel_k2c=L0_prel_k2c)
    L[1].update(k_W_cpd=L1_k_W_cpd, k_b_cpd=L1_k_b_cpd, q_W_cpd=L1_q_W_cpd,
                q_b_cpd=L1_q_b_cpd, v_W_cpd=L1_v_W_cpd, v_b_cpd=L1_v_b_cpd,
                a_W_cpd=L1_a_W_cpd, a_b_cpd=L1_a_b_cpd, skip_cpd=L1_skip_cpd,
                k_W_ko=L1_k_W_ko, k_b_ko=L1_k_b_ko, q_W_ko=L1_q_W_ko,
                q_b_ko=L1_q_b_ko, v_W_ko=L1_v_W_ko, v_b_ko=L1_v_b_ko,
                a_W_ko=L1_a_W_ko, a_b_ko=L1_a_b_ko, skip_ko=L1_skip_ko,
                arel_c2k=L1_arel_c2k, mrel_c2k=L1_mrel_c2k,
                prel_c2k=L1_prel_c2k, arel_k2c=L1_arel_k2c,
                mrel_k2c=L1_mrel_k2c, prel_k2c=L1_prel_k2c)

    src_g_c2k, dst_g_c2k, dst_s_c2k = _pad_edges(edge_index_cpd_to_ko)
    src_g_k2c, dst_g_k2c, dst_s_k2c = _pad_edges(edge_index_ko_to_cpd)
    zeros_v = jnp.zeros((ACC_ROWS, HID), jnp.float32)
    zeros_e = jnp.zeros((ACC_ROWS, EW), jnp.float32)
    hsum = jnp.kron(jnp.eye(HEADS, dtype=jnp.float32),
                    jnp.ones((DH, DH), jnp.float32))
    sel = jnp.kron(jnp.eye(HEADS, dtype=jnp.float32),
                   jnp.ones((DH, 1), jnp.float32))
    sel = sel * (jnp.arange(HID) % DH == 0).astype(jnp.float32)[:, None]
    bc = jnp.zeros((EW, HID), jnp.float32).at[:HEADS].set(
        jnp.kron(jnp.eye(HEADS, dtype=jnp.float32),
                 jnp.ones((1, DH), jnp.float32)))

    h_cpd = _linear(_pad_nodes(x_cpd), lin0_W_cpd, lin0_b_cpd, "relu", 1568)
    h_ko = _linear(_pad_nodes(x_ko), lin0_W_ko, lin0_b_ko, "relu", 1568)

    for l in (0, 1):
        P = L[l]
        wcat_c, bcat_c = _combined_weights(
            P["q_W_cpd"], P["q_b_cpd"], P["k_W_cpd"], P["k_b_cpd"],
            P["v_W_cpd"], P["v_b_cpd"],
            P["arel_c2k"], P["mrel_c2k"], P["prel_c2k"])
        wcat_k, bcat_k = _combined_weights(
            P["q_W_ko"], P["q_b_ko"], P["k_W_ko"], P["k_b_ko"],
            P["v_W_ko"], P["v_b_ko"],
            P["arel_k2c"], P["mrel_k2c"], P["prel_k2c"])
        q_c, kv_c = _proj(h_cpd, wcat_c, bcat_c)
        q_k, kv_k = _proj(h_ko, wcat_k, bcat_k)

        qg1, kvg1 = _gather(q_k, kv_c, dst_g_c2k, src_g_c2k)
        mv1, me1 = _msg(qg1, kvg1, hsum, sel)
        num_ko = _scatter(mv1, dst_s_c2k, zeros_v, HID)
        den_ko = _scatter(me1, dst_s_c2k, zeros_e, EW)

        qg2, kvg2 = _gather(q_c, kv_k, dst_g_k2c, src_g_k2c)
        mv2, me2 = _msg(qg2, kvg2, hsum, sel)
        num_cpd = _scatter(mv2, dst_s_k2c, zeros_v, HID)
        den_cpd = _scatter(me2, dst_s_k2c, zeros_e, EW)

        h_cpd = _update(num_cpd, den_cpd, bc, h_cpd, P["a_W_cpd"],
                        P["a_b_cpd"], P["skip_cpd"])
        h_ko = _update(num_ko, den_ko, bc, h_ko, P["a_W_ko"], P["a_b_ko"],
                       P["skip_ko"])

    out_cpd = _linear(h_cpd, lin_out_W, lin_out_b, "none", 1000,
                      m_out=N_REAL)
    out_ko = _linear(h_ko, lin_out_W, lin_out_b, "none", 1000, m_out=N_REAL)
    return (out_cpd, out_ko)


# speedup vs baseline: 38.2851x; 1.1409x over previous
import functools
import math

import jax
import jax.numpy as jnp
from jax import lax
from jax.experimental import pallas as pl
from jax.experimental.pallas import tpu as pltpu, tpu_sc as plsc
from jax.scipy.linalg import block_diag

N_REAL = 25000
D_IN = 128
HID = 128
HEADS = 4
DH = HID // HEADS
OUT = 64
E_REAL = 300000

NP = 25088
HALF = 12544
ACC_ROWS = 12672
EP = 303104
EW = 16

_sc_mesh = plsc.VectorSubcoreMesh(core_axis_name="c", subcore_axis_name="s")
_sc_params = pltpu.CompilerParams(use_tc_tiling_on_sc=False,
                                  needs_layout_passes=False)


def _linear_body(act, x_ref, w_ref, b_ref, o_ref):
    y = jnp.dot(x_ref[...], w_ref[...], preferred_element_type=jnp.float32)
    y = y + b_ref[...]
    if act == "relu":
        y = jnp.maximum(y, 0.0)
    o_ref[...] = y


def _linear(x, w, b, act, block, m_out=None):
    m = m_out if m_out is not None else x.shape[0]
    k = x.shape[1]
    n = w.shape[1]
    return pl.pallas_call(
        functools.partial(_linear_body, act),
        grid=(m // block,),
        in_specs=[
            pl.BlockSpec((block, k), lambda i: (i, 0)),
            pl.BlockSpec((k, n), lambda i: (0, 0)),
            pl.BlockSpec((1, n), lambda i: (0, 0)),
        ],
        out_specs=pl.BlockSpec((block, n), lambda i: (i, 0)),
        out_shape=jax.ShapeDtypeStruct((m, n), jnp.float32),
    )(x, w, b.reshape(1, n))


def _proj_body(x_ref, w_ref, b_ref, q_ref, kv_ref):
    y = jnp.dot(x_ref[...], w_ref[...], preferred_element_type=jnp.float32)
    y = y + b_ref[...]
    q_ref[...] = y[:, :HID]
    kv_ref[...] = y[:, HID:]


def _proj(h, wcat, bcat, block=1568):
    return pl.pallas_call(
        _proj_body,
        grid=(NP // block,),
        in_specs=[
            pl.BlockSpec((block, HID), lambda i: (i, 0)),
            pl.BlockSpec((HID, 3 * HID), lambda i: (0, 0)),
            pl.BlockSpec((1, 3 * HID), lambda i: (0, 0)),
        ],
        out_specs=[
            pl.BlockSpec((block, HID), lambda i: (i, 0)),
            pl.BlockSpec((block, 2 * HID), lambda i: (i, 0)),
        ],
        out_shape=[
            jax.ShapeDtypeStruct((NP, HID), jnp.float32),
            jax.ShapeDtypeStruct((NP, 2 * HID), jnp.float32),
        ],
    )(h, wcat, bcat.reshape(1, 3 * HID))


def _msg_body(q_ref, kv_ref, hsum_ref, sel_ref, v_ref, e_ref):
    q = q_ref[...]
    kv = kv_ref[...]
    k = kv[:, :HID]
    v = kv[:, HID:]
    p = q * k
    s128 = jnp.dot(p, hsum_ref[...], preferred_element_type=jnp.float32)
    e128 = jnp.exp(s128)
    e4 = jnp.dot(e128, sel_ref[...], preferred_element_type=jnp.float32)
    z = jnp.zeros((q.shape[0], EW - HEADS), jnp.float32)
    v_ref[...] = v * e128
    e_ref[...] = jnp.concatenate([e4, z], axis=1)


def _msg(qg, kvg, hsum, sel, block=2048):
    return pl.pallas_call(
        _msg_body,
        grid=(EP // block,),
        in_specs=[
            pl.BlockSpec((block, HID), lambda i: (i, 0)),
            pl.BlockSpec((block, 2 * HID), lambda i: (i, 0)),
            pl.BlockSpec((HID, HID), lambda i: (0, 0)),
            pl.BlockSpec((HID, HEADS), lambda i: (0, 0)),
        ],
        out_specs=[
            pl.BlockSpec((block, HID), lambda i: (i, 0)),
            pl.BlockSpec((block, EW), lambda i: (i, 0)),
        ],
        out_shape=[
            jax.ShapeDtypeStruct((EP, HID), jnp.float32),
            jax.ShapeDtypeStruct((EP, EW), jnp.float32),
        ],
    )(qg, kvg, hsum, sel)


def _update_body(num_ref, den_ref, bc_ref, h_ref, w_ref, b_ref, s_ref, o_ref):
    den = jnp.dot(den_ref[...], bc_ref[...],
                  preferred_element_type=jnp.float32)
    agg = num_ref[...] / (den + 1e-16)
    g = jax.nn.gelu(agg)
    o = jnp.dot(g, w_ref[...], preferred_element_type=jnp.float32) + b_ref[...]
    a = jax.nn.sigmoid(s_ref[0, 0])
    o_ref[...] = a * o + (1.0 - a) * h_ref[...]


def _update(num, den, bc, h, w, b, skip, block=1568):
    return pl.pallas_call(
        _update_body,
        grid=(NP // block,),
        in_specs=[
            pl.BlockSpec((block, HID), lambda i: (i, 0)),
            pl.BlockSpec((block, EW), lambda i: (i, 0)),
            pl.BlockSpec((EW, HID), lambda i: (0, 0)),
            pl.BlockSpec((block, HID), lambda i: (i, 0)),
            pl.BlockSpec((HID, HID), lambda i: (0, 0)),
            pl.BlockSpec((1, HID), lambda i: (0, 0)),
            pl.BlockSpec((1, 1), lambda i: (0, 0)),
        ],
        out_specs=pl.BlockSpec((block, HID), lambda i: (i, 0)),
        out_shape=jax.ShapeDtypeStruct((NP, HID), jnp.float32),
    )(num, den, bc, h, w, b.reshape(1, HID), skip.reshape(1, 1))


def _gather(q_tab, kv_tab, dst_g, src_g):
    epw = EP // 32
    ch = 128
    nch = epw // ch

    @functools.partial(
        pl.kernel,
        mesh=_sc_mesh,
        compiler_params=_sc_params,
        out_type=[
            jax.ShapeDtypeStruct((EP, HID), jnp.float32),
            jax.ShapeDtypeStruct((EP, 2 * HID), jnp.float32),
        ],
        scratch_types=[
            pltpu.VMEM((epw,), jnp.int32),
            pltpu.VMEM((epw,), jnp.int32),
            pltpu.VMEM((ch, HID), jnp.float32),
            pltpu.VMEM((ch, HID), jnp.float32),
            pltpu.VMEM((ch, 2 * HID), jnp.float32),
            pltpu.VMEM((ch, 2 * HID), jnp.float32),
        ] + [pltpu.SemaphoreType.DMA] * 8,
    )
    def k(qt, kvt, dg, sg, qg, kvg, dib, sib, qb0, qb1, kvb0, kvb1,
          gq0, gq1, gk0, gk1, wq0, wq1, wk0, wk1):
        wid = lax.axis_index("s") * 2 + lax.axis_index("c")
        base = wid * epw
        pltpu.sync_copy(dg.at[pl.ds(base, epw)], dib)
        pltpu.sync_copy(sg.at[pl.ds(base, epw)], sib)

        def issue_g(off, qb, kvb, sq, sk):
            pltpu.async_copy(qt.at[dib.at[pl.ds(off, ch)]], qb, sq)
            pltpu.async_copy(kvt.at[sib.at[pl.ds(off, ch)]], kvb, sk)

        def wait_g(qb, kvb, sq, sk):
            pltpu.make_async_copy(qt.at[dib.at[pl.ds(0, ch)]], qb, sq).wait()
            pltpu.make_async_copy(kvt.at[sib.at[pl.ds(0, ch)]], kvb, sk).wait()

        def issue_w(off, qb, kvb, sq, sk):
            pltpu.async_copy(qb, qg.at[pl.ds(base + off, ch)], sq)
            pltpu.async_copy(kvb, kvg.at[pl.ds(base + off, ch)], sk)

        def wait_w(qb, kvb, sq, sk):
            pltpu.make_async_copy(qb, qg.at[pl.ds(base, ch)], sq).wait()
            pltpu.make_async_copy(kvb, kvg.at[pl.ds(base, ch)], sk).wait()

        issue_g(0, qb0, kvb0, gq0, gk0)

        def body(i, carry):
            c0 = 2 * i * ch
            c1 = c0 + ch
            c2 = c1 + ch

            @pl.when(i > 0)
            def _():
                wait_w(qb1, kvb1, wq1, wk1)

            issue_g(c1, qb1, kvb1, gq1, gk1)
            wait_g(qb0, kvb0, gq0, gk0)
            issue_w(c0, qb0, kvb0, wq0, wk0)
            wait_w(qb0, kvb0, wq0, wk0)

            @pl.when(i < nch // 2 - 1)
            def _():
                issue_g(c2, qb0, kvb0, gq0, gk0)

            wait_g(qb1, kvb1, gq1, gk1)
            issue_w(c1, qb1, kvb1, wq1, wk1)
            return carry

        lax.fori_loop(0, nch // 2, body, 0)
        wait_w(qb1, kvb1, wq1, wk1)

    return k(q_tab, kv_tab, dst_g, src_g)


def _scatter(msg, dst_s, zeros_acc, width, ch):
    ept = EP // 16
    nch = ept // ch
    zrows = ACC_ROWS // 16
    rpt = HALF // 16

    @functools.partial(
        pl.kernel,
        mesh=_sc_mesh,
        compiler_params=_sc_params,
        out_type=jax.ShapeDtypeStruct((NP, width), jnp.float32),
        scratch_types=[
            pltpu.VMEM((ch,), jnp.int32),
            pltpu.VMEM((ch,), jnp.int32),
            pltpu.VMEM((ch,), jnp.int32),
            pltpu.VMEM((ch,), jnp.int32),
            pltpu.VMEM((ch, width), jnp.float32),
            pltpu.VMEM((ch, width), jnp.float32),
            pltpu.VMEM_SHARED((ACC_ROWS, width), jnp.float32),
        ] + [pltpu.SemaphoreType.DMA] * 6,
    )
    def k(mh, dh, zh, agg, di0, di1, li0, li1, mb0, mb1, acc,
          sd0, sd1, sm0, sm1, sa0, sa1):
        cid = lax.axis_index("c")
        sid = lax.axis_index("s")
        pltpu.sync_copy(zh.at[pl.ds(sid * zrows, zrows)],
                        acc.at[pl.ds(sid * zrows, zrows)])

        nbase = cid * HALF
        ebase = sid * ept
        plsc.subcore_barrier()

        def issue_in(ci, di, mb, sd, sm):
            pltpu.async_copy(dh.at[pl.ds(ebase + ci * ch, ch)], di, sd)
            pltpu.async_copy(mh.at[pl.ds(ebase + ci * ch, ch)], mb, sm)

        def wait_in(di, mb, sd, sm):
            pltpu.make_async_copy(dh.at[pl.ds(ebase, ch)], di, sd).wait()
            pltpu.make_async_copy(mh.at[pl.ds(ebase, ch)], mb, sm).wait()

        def comp_li(di, li):
            for g in range(ch // 16):
                dv = di[pl.ds(g * 16, 16)]
                l = dv - nbase
                ok = (l >= 0) & (l < HALF)
                li[pl.ds(g * 16, 16)] = jnp.where(ok, l, HALF)

        def issue_a(mb, li, sem):
            pltpu.async_copy(mb, acc.at[li], sem, add=True)

        def wait_a(mb, li, sem):
            pltpu.make_async_copy(mb, acc.at[li], sem).wait()

        issue_in(0, di0, mb0, sd0, sm0)

        def body(i, carry):
            c1 = 2 * i + 1
            c2 = 2 * i + 2

            @pl.when(i > 0)
            def _():
                wait_a(mb1, li1, sa1)

            issue_in(c1, di1, mb1, sd1, sm1)
            wait_in(di0, mb0, sd0, sm0)
            comp_li(di0, li0)
            issue_a(mb0, li0, sa0)
            wait_a(mb0, li0, sa0)

            @pl.when(i < nch // 2 - 1)
            def _():
                issue_in(c2, di0, mb0, sd0, sm0)

            wait_in(di1, mb1, sd1, sm1)
            comp_li(di1, li1)
            issue_a(mb1, li1, sa1)
            return carry

        lax.fori_loop(0, nch // 2, body, 0)
        wait_a(mb1, li1, sa1)
        plsc.subcore_barrier()

        pltpu.sync_copy(acc.at[pl.ds(sid * rpt, rpt)],
                        agg.at[pl.ds(nbase + sid * rpt, rpt)])

    return k(msg, dst_s, zeros_acc)


def _pad_nodes(x):
    return jnp.concatenate(
        [x, jnp.zeros((NP - N_REAL, x.shape[1]), x.dtype)], axis=0)


def _pad_edges(ei):
    src, dst = ei[0], ei[1]
    pad = EP - E_REAL
    src_g = jnp.concatenate([src, jnp.zeros((pad,), jnp.int32)])
    dst_g = jnp.concatenate([dst, jnp.zeros((pad,), jnp.int32)])
    dst_s = jnp.concatenate([dst, jnp.full((pad,), NP, jnp.int32)])
    return src_g, dst_g, dst_s


def _combined_weights(wq, bq, wk, bk, wv, bv, arel, mrel, prel):
    scale = prel / math.sqrt(DH)
    arel_s = arel * scale[:, None, None]
    bd_a = block_diag(*[arel_s[h] for h in range(HEADS)])
    bd_m = block_diag(*[mrel[h] for h in range(HEADS)])
    wcat = jnp.concatenate([wq, wk @ bd_a, wv @ bd_m], axis=1)
    bcat = jnp.concatenate([bq, bk @ bd_a, bv @ bd_m], axis=0)
    return wcat, bcat


def kernel(x_cpd, x_ko, edge_index_cpd_to_ko, edge_index_ko_to_cpd,
           lin0_W_cpd, lin0_b_cpd, lin0_W_ko, lin0_b_ko,
           L0_k_W_cpd, L0_k_b_cpd, L0_q_W_cpd, L0_q_b_cpd, L0_v_W_cpd,
           L0_v_b_cpd, L0_a_W_cpd, L0_a_b_cpd, L0_skip_cpd,
           L0_k_W_ko, L0_k_b_ko, L0_q_W_ko, L0_q_b_ko, L0_v_W_ko, L0_v_b_ko,
           L0_a_W_ko, L0_a_b_ko, L0_skip_ko,
           L0_arel_c2k, L0_mrel_c2k, L0_prel_c2k,
           L0_arel_k2c, L0_mrel_k2c, L0_prel_k2c,
           L1_k_W_cpd, L1_k_b_cpd, L1_q_W_cpd, L1_q_b_cpd, L1_v_W_cpd,
           L1_v_b_cpd, L1_a_W_cpd, L1_a_b_cpd, L1_skip_cpd,
           L1_k_W_ko, L1_k_b_ko, L1_q_W_ko, L1_q_b_ko, L1_v_W_ko, L1_v_b_ko,
           L1_a_W_ko, L1_a_b_ko, L1_skip_ko,
           L1_arel_c2k, L1_mrel_c2k, L1_prel_c2k,
           L1_arel_k2c, L1_mrel_k2c, L1_prel_k2c,
           lin_out_W, lin_out_b):
    L = {0: {}, 1: {}}
    L[0].update(k_W_cpd=L0_k_W_cpd, k_b_cpd=L0_k_b_cpd, q_W_cpd=L0_q_W_cpd,
                q_b_cpd=L0_q_b_cpd, v_W_cpd=L0_v_W_cpd, v_b_cpd=L0_v_b_cpd,
                a_W_cpd=L0_a_W_cpd, a_b_cpd=L0_a_b_cpd, skip_cpd=L0_skip_cpd,
                k_W_ko=L0_k_W_ko, k_b_ko=L0_k_b_ko, q_W_ko=L0_q_W_ko,
                q_b_ko=L0_q_b_ko, v_W_ko=L0_v_W_ko, v_b_ko=L0_v_b_ko,
                a_W_ko=L0_a_W_ko, a_b_ko=L0_a_b_ko, skip_ko=L0_skip_ko,
                arel_c2k=L0_arel_c2k, mrel_c2k=L0_mrel_c2k,
                prel_c2k=L0_prel_c2k, arel_k2c=L0_arel_k2c,
                mrel_k2c=L0_mrel_k2c, prel_k2c=L0_prel_k2c)
    L[1].update(k_W_cpd=L1_k_W_cpd, k_b_cpd=L1_k_b_cpd, q_W_cpd=L1_q_W_cpd,
                q_b_cpd=L1_q_b_cpd, v_W_cpd=L1_v_W_cpd, v_b_cpd=L1_v_b_cpd,
                a_W_cpd=L1_a_W_cpd, a_b_cpd=L1_a_b_cpd, skip_cpd=L1_skip_cpd,
                k_W_ko=L1_k_W_ko, k_b_ko=L1_k_b_ko, q_W_ko=L1_q_W_ko,
                q_b_ko=L1_q_b_ko, v_W_ko=L1_v_W_ko, v_b_ko=L1_v_b_ko,
                a_W_ko=L1_a_W_ko, a_b_ko=L1_a_b_ko, skip_ko=L1_skip_ko,
                arel_c2k=L1_arel_c2k, mrel_c2k=L1_mrel_c2k,
                prel_c2k=L1_prel_c2k, arel_k2c=L1_arel_k2c,
                mrel_k2c=L1_mrel_k2c, prel_k2c=L1_prel_k2c)

    src_g_c2k, dst_g_c2k, dst_s_c2k = _pad_edges(edge_index_cpd_to_ko)
    src_g_k2c, dst_g_k2c, dst_s_k2c = _pad_edges(edge_index_ko_to_cpd)
    zeros_v = jnp.zeros((ACC_ROWS, HID), jnp.float32)
    zeros_e = jnp.zeros((ACC_ROWS, EW), jnp.float32)
    hsum = jnp.kron(jnp.eye(HEADS, dtype=jnp.float32),
                    jnp.ones((DH, DH), jnp.float32))
    sel = jnp.kron(jnp.eye(HEADS, dtype=jnp.float32),
                   jnp.ones((DH, 1), jnp.float32))
    sel = sel * (jnp.arange(HID) % DH == 0).astype(jnp.float32)[:, None]
    bc = jnp.zeros((EW, HID), jnp.float32).at[:HEADS].set(
        jnp.kron(jnp.eye(HEADS, dtype=jnp.float32),
                 jnp.ones((1, DH), jnp.float32)))

    h_cpd = _linear(_pad_nodes(x_cpd), lin0_W_cpd, lin0_b_cpd, "relu", 1568)
    h_ko = _linear(_pad_nodes(x_ko), lin0_W_ko, lin0_b_ko, "relu", 1568)

    for l in (0, 1):
        P = L[l]
        wcat_c, bcat_c = _combined_weights(
            P["q_W_cpd"], P["q_b_cpd"], P["k_W_cpd"], P["k_b_cpd"],
            P["v_W_cpd"], P["v_b_cpd"],
            P["arel_c2k"], P["mrel_c2k"], P["prel_c2k"])
        wcat_k, bcat_k = _combined_weights(
            P["q_W_ko"], P["q_b_ko"], P["k_W_ko"], P["k_b_ko"],
            P["v_W_ko"], P["v_b_ko"],
            P["arel_k2c"], P["mrel_k2c"], P["prel_k2c"])
        q_c, kv_c = _proj(h_cpd, wcat_c, bcat_c)
        q_k, kv_k = _proj(h_ko, wcat_k, bcat_k)

        qg1, kvg1 = _gather(q_k, kv_c, dst_g_c2k, src_g_c2k)
        mv1, me1 = _msg(qg1, kvg1, hsum, sel)
        num_ko = _scatter(mv1, dst_s_c2k, zeros_v, HID, 64)
        den_ko = _scatter(me1, dst_s_c2k, zeros_e, EW, 128)

        qg2, kvg2 = _gather(q_c, kv_k, dst_g_k2c, src_g_k2c)
        mv2, me2 = _msg(qg2, kvg2, hsum, sel)
        num_cpd = _scatter(mv2, dst_s_k2c, zeros_v, HID, 64)
        den_cpd = _scatter(me2, dst_s_k2c, zeros_e, EW, 128)

        h_cpd = _update(num_cpd, den_cpd, bc, h_cpd, P["a_W_cpd"],
                        P["a_b_cpd"], P["skip_cpd"])
        h_ko = _update(num_ko, den_ko, bc, h_ko, P["a_W_ko"], P["a_b_ko"],
                       P["skip_ko"])

    out_cpd = _linear(h_cpd, lin_out_W, lin_out_b, "none", 1000,
                      m_out=N_REAL)
    out_ko = _linear(h_ko, lin_out_W, lin_out_b, "none", 1000, m_out=N_REAL)
    return (out_cpd, out_ko)
